# baseline (speedup 1.0000x reference)
import jax
import jax.numpy as jnp
from jax.experimental import pallas as pl
from jax.experimental.pallas import tpu as pltpu

_B, _L, _D, _H, _U, _OUT, _HD, _K = 16, 1024, 256, 256, 8, 3, 512, 6
_SPAN = _L // _U
_OPAD = 128


def _mm(a, b):
    return jax.lax.dot_general(a, b, (((1,), (0,)), ((), ())),
                               preferred_element_type=jnp.float32)


def _mmT(a, b):
    return jax.lax.dot_general(a, b, (((0,), (0,)), ((), ())),
                               preferred_element_type=jnp.float32)


def _sig(x):
    return 0.5 * jnp.tanh(0.5 * x) + 0.5


def _body(x_ref, te_ref, deps_ref, roots_ref, mask_ref,
          Wiou_ref, biou_ref, Wf_ref, UfUiou_ref, bf_ref,
          W1_ref, b1_ref, W2_ref, b2_ref, out_ref):
    f32 = jnp.float32
    bf16 = jnp.bfloat16
    L, H, D, HD, U, K = _L, _H, _D, _HD, _U, _K

    x = x_ref[0]
    te = te_ref[0]
    deps = deps_ref[0]
    mask = mask_ref[0]
    root = roots_ref[0]

    Wiou = Wiou_ref[...]
    biou = biou_ref[...]
    Wf = Wf_ref[...]
    bf = bf_ref[...]
    UfUiou = UfUiou_ref[...]
    Uf = UfUiou[:, :H]
    Uiou = UfUiou[:, H:]
    W1 = W1_ref[...]
    b1 = b1_ref[...]
    W2 = W2_ref[...]
    b2 = b2_ref[...]

    iota_row = jax.lax.broadcasted_iota(jnp.int32, (1, L), 1)
    v_row = ((deps != iota_row) & (mask > 0.5)).astype(f32)
    row_ids = jax.lax.broadcasted_iota(jnp.int32, (L, L), 0)
    Tv = ((row_ids == deps).astype(f32) * v_row).astype(bf16)
    valid_col = _mmT(Tv, jnp.ones((L, 1), bf16))

    xiou = _mm(x, Wiou) + biou
    xf = _mm(x, Wf) + bf
    xf_dst = _mmT(Tv, xf.astype(bf16))

    def _gates(iou, fc):
        i = _sig(iou[:, :H])
        o = _sig(iou[:, H:2 * H])
        u = jnp.tanh(iou[:, 2 * H:])
        c = i * u + fc
        h = o * jnp.tanh(c)
        return h, c

    def both_rounds(carry):
        hb, cb, ht, ct = carry
        hb_bf = hb.astype(bf16)
        h_sum = _mm(Tv, hb_bf)
        f_b = _sig(xf_dst + _mm(hb_bf, Uf) * valid_col)
        fc_sum = _mm(Tv, (f_b * cb).astype(bf16))
        g = _mmT(Tv, jnp.concatenate([ht.astype(bf16), ct.astype(bf16)], axis=1))
        fused = _mm(g[:, :H].astype(bf16), UfUiou)
        iou_b = xiou + _mm(h_sum.astype(bf16), Uiou)
        hb, cb = _gates(iou_b, fc_sum)
        f_t = _sig(xf + fused[:, :H])
        ht, ct = _gates(xiou + fused[:, H:], f_t * g[:, H:])
        return hb, cb, ht, ct

    z = jnp.zeros((L, H), f32)
    carry = (z, z, z, z)
    for _ in range(K):
        carry = both_rounds(carry)
    h_bu, _, h_td, _ = carry

    r_oh = (iota_row == root).astype(f32)
    HpA = _mm(r_oh, h_bu)
    kk = jax.lax.broadcasted_iota(jnp.int32, (U, L), 0)
    jj = jax.lax.broadcasted_iota(jnp.int32, (U, L), 1)
    P1 = (jj == kk * _SPAN).astype(f32)
    P2 = (jj == kk * _SPAN + (_SPAN - 1)).astype(f32)
    M8 = ((jj // _SPAN) == kk).astype(f32) * (1.0 / _SPAN)
    Hp1 = _mm(P1, h_td)
    Hp2 = _mm(P2, h_td)
    avg = _mm(M8, te)

    g0 = _mm(HpA, W1[0:H]) + b1
    G1 = _mm(Hp1, W1[H:2 * H])
    G2 = _mm(Hp2, W1[2 * H:3 * H])
    A1 = _mm(avg, W1[3 * H:3 * H + D])
    A2 = _mm(avg, W1[3 * H + D:3 * H + 2 * D])

    rowt = (G1 + A1).reshape(U, 1, HD)
    colt = (G2 + A2).reshape(1, U, HD)
    hid = jnp.tanh(rowt + colt + g0.reshape(1, 1, HD))
    out = _mm(hid.reshape(U * U, HD), W2) + b2
    out_ref[0] = out


def kernel(node_embs, token_embs, dependencies, roots, token_mask,
           W_iou, U_iou, b_iou, W_f, U_f, b_f, W1, b1, W2, b2):
    B, L, D, H, HD = _B, _L, _D, _H, _HD
    deps3 = dependencies.astype(jnp.int32).reshape(B, 1, L)
    roots3 = roots.astype(jnp.int32).reshape(B, 1, 1)
    mask3 = token_mask.astype(jnp.float32).reshape(B, 1, L)
    biou2 = b_iou.reshape(1, 3 * H)
    bf2 = b_f.reshape(1, H)
    b1_2 = b1.reshape(1, HD)
    W2p = jnp.pad(W2, ((0, 0), (0, _OPAD - _OUT)))
    b2p = jnp.pad(b2, (0, _OPAD - _OUT)).reshape(1, _OPAD)
    UfUiou = jnp.concatenate([U_f, U_iou], axis=1).astype(jnp.bfloat16)

    const = lambda b: (0, 0)
    per_b3 = lambda b: (b, 0, 0)
    in_specs = [
        pl.BlockSpec((1, L, D), per_b3),
        pl.BlockSpec((1, L, D), per_b3),
        pl.BlockSpec((1, 1, L), per_b3),
        pl.BlockSpec((1, 1, 1), per_b3),
        pl.BlockSpec((1, 1, L), per_b3),
        pl.BlockSpec((D, 3 * H), const),
        pl.BlockSpec((1, 3 * H), const),
        pl.BlockSpec((D, H), const),
        pl.BlockSpec((H, 4 * H), const),
        pl.BlockSpec((1, H), const),
        pl.BlockSpec((3 * H + 2 * D, HD), const),
        pl.BlockSpec((1, HD), const),
        pl.BlockSpec((HD, _OPAD), const),
        pl.BlockSpec((1, _OPAD), const),
    ]
    out = pl.pallas_call(
        _body,
        grid=(B,),
        in_specs=in_specs,
        out_specs=pl.BlockSpec((1, _U * _U, _OPAD), per_b3),
        out_shape=jax.ShapeDtypeStruct((B, _U * _U, _OPAD), jnp.float32),
        compiler_params=pltpu.CompilerParams(
            dimension_semantics=("parallel",),
            allow_input_fusion=[False] * 2 + [True] * 3 + [False] * 3 + [True] * 1
            + [False] * 1 + [False] * 2 + [True] * 2),
    )(node_embs, token_embs, deps3, roots3, mask3,
      W_iou, biou2, W_f, UfUiou, bf2, W1, b1_2, W2p, b2p)
    return out[:, :, :_OUT].reshape(B, _U, _U, _OUT)


# speedup vs baseline: 1.2095x; 1.2095x over previous
import jax
import jax.numpy as jnp
from jax.experimental import pallas as pl
from jax.experimental.pallas import tpu as pltpu

_B, _L, _D, _H, _U, _OUT, _HD, _K = 16, 1024, 256, 256, 8, 3, 512, 6
_SPAN = _L // _U
_OPAD = 128


def _mm(a, b):
    return jax.lax.dot_general(a, b, (((1,), (0,)), ((), ())),
                               preferred_element_type=jnp.float32)


def _mmT(a, b):
    return jax.lax.dot_general(a, b, (((0,), (0,)), ((), ())),
                               preferred_element_type=jnp.float32)


def _sig(x):
    return 0.5 * jnp.tanh(0.5 * x) + 0.5


def _body(x_ref, te_ref, deps_ref, roots_ref, mask_ref,
          Wiou_ref, biou_ref, Wf_ref, UfUiou_ref, bf_ref,
          W1_ref, b1_ref, W2_ref, b2_ref, out_ref):
    f32 = jnp.float32
    bf16 = jnp.bfloat16
    L, H, D, HD, U, K = _L, _H, _D, _HD, _U, _K

    x = x_ref[0]
    te = te_ref[0]
    deps = deps_ref[0]
    mask = mask_ref[0]
    root = roots_ref[0]

    Wiou = Wiou_ref[...]
    biou = biou_ref[...]
    Wf = Wf_ref[...]
    bf = bf_ref[...]
    UfUiou = UfUiou_ref[...]
    Uf = UfUiou[:, :H]
    Uiou = UfUiou[:, H:]
    W1 = W1_ref[...]
    b1 = b1_ref[...]
    W2 = W2_ref[...]
    b2 = b2_ref[...]

    iota_row = jax.lax.broadcasted_iota(jnp.int32, (1, L), 1)
    v_row = ((deps != iota_row) & (mask > 0.5)).astype(f32)
    row_ids = jax.lax.broadcasted_iota(jnp.int32, (L, L), 0)
    Tv = ((row_ids == deps).astype(f32) * v_row).astype(bf16)
    valid_col = _mmT(Tv, jnp.ones((L, 1), bf16))

    xiou = _mm(x, Wiou) + biou
    xf = _mm(x, Wf) + bf
    xf_dst = _mmT(Tv, xf.astype(bf16))

    def _gates(iou, fc):
        i = _sig(iou[:, :H])
        o = _sig(iou[:, H:2 * H])
        u = jnp.tanh(iou[:, 2 * H:])
        c = i * u + fc
        h = o * jnp.tanh(c)
        return h, c

    def both_rounds(carry):
        hb, cb, ht, ct = carry
        hb_bf = hb.astype(bf16)
        f_b = _sig(xf_dst + _mm(hb_bf, Uf) * valid_col)
        sc = _mm(Tv, jnp.concatenate([hb_bf, (f_b * cb).astype(bf16)], axis=1))
        g = _mmT(Tv, jnp.concatenate([ht.astype(bf16), ct.astype(bf16)], axis=1))
        fused = _mm(g[:, :H].astype(bf16), UfUiou)
        iou_b = xiou + _mm(sc[:, :H].astype(bf16), Uiou)
        hb, cb = _gates(iou_b, sc[:, H:])
        f_t = _sig(xf + fused[:, :H])
        ht, ct = _gates(xiou + fused[:, H:], f_t * g[:, H:])
        return hb, cb, ht, ct

    h1, c1 = _gates(xiou, jnp.zeros((L, H), f32))
    carry = (h1, c1, h1, c1)
    for _ in range(K - 2):
        carry = both_rounds(carry)
    hb, cb, ht, ct = carry

    r_oh = (iota_row == root).astype(f32)
    hb_bf = hb.astype(bf16)
    f_b = _sig(xf_dst + _mm(hb_bf, Uf) * valid_col)
    A = jnp.concatenate([hb_bf, (f_b * cb).astype(bf16)], axis=1)
    w_row = ((deps == root).astype(f32) * v_row).astype(bf16)
    sc_r = _mm(w_row, A)
    iou_r = _mm(r_oh, xiou) + _mm(sc_r[:, :H].astype(bf16), Uiou)
    HpA, _ = _gates(iou_r, sc_r[:, H:])

    kk16 = jax.lax.broadcasted_iota(jnp.int32, (2 * U, L), 0)
    jj16 = jax.lax.broadcasted_iota(jnp.int32, (2 * U, L), 1)
    P16 = (jj16 == jnp.where(kk16 < U, kk16 * _SPAN,
                             (kk16 - U) * _SPAN + (_SPAN - 1))).astype(f32)
    jcol = jax.lax.broadcasted_iota(jnp.int32, (L, 2), 1)
    icol = jax.lax.broadcasted_iota(jnp.int32, (L, 2), 0)
    jparts = jnp.where(jcol == 0, icol // 4, icol % 4).astype(bf16)
    dep2 = _mmT(Tv, jparts)
    dep_rows2 = _mm(P16, dep2)
    dep_rows = 4.0 * dep_rows2[:, :1] + dep_rows2[:, 1:]
    valid_rows = _mm(P16, valid_col)
    Qg = ((jj16.astype(f32) == dep_rows) * valid_rows).astype(bf16)
    g16 = _mm(Qg, jnp.concatenate([ht.astype(bf16), ct.astype(bf16)], axis=1))
    fused16 = _mm(g16[:, :H].astype(bf16), UfUiou)
    xiou16 = _mm(P16, xiou)
    xf16 = _mm(P16, xf)
    f_t16 = _sig(xf16 + fused16[:, :H])
    h16, _ = _gates(xiou16 + fused16[:, H:], f_t16 * g16[:, H:])
    Hp1 = h16[:U]
    Hp2 = h16[U:]
    kk = jax.lax.broadcasted_iota(jnp.int32, (U, L), 0)
    jj = jax.lax.broadcasted_iota(jnp.int32, (U, L), 1)
    M8 = ((jj // _SPAN) == kk).astype(f32) * (1.0 / _SPAN)
    avg = _mm(M8, te)

    g0 = _mm(HpA, W1[0:H]) + b1
    G1 = _mm(Hp1, W1[H:2 * H])
    G2 = _mm(Hp2, W1[2 * H:3 * H])
    A1 = _mm(avg, W1[3 * H:3 * H + D])
    A2 = _mm(avg, W1[3 * H + D:3 * H + 2 * D])

    rowt = (G1 + A1).reshape(U, 1, HD)
    colt = (G2 + A2).reshape(1, U, HD)
    hid = jnp.tanh(rowt + colt + g0.reshape(1, 1, HD))
    out = _mm(hid.reshape(U * U, HD), W2) + b2
    out_ref[0] = out


def kernel(node_embs, token_embs, dependencies, roots, token_mask,
           W_iou, U_iou, b_iou, W_f, U_f, b_f, W1, b1, W2, b2):
    B, L, D, H, HD = _B, _L, _D, _H, _HD
    deps3 = dependencies.astype(jnp.int32).reshape(B, 1, L)
    roots3 = roots.astype(jnp.int32).reshape(B, 1, 1)
    mask3 = token_mask.astype(jnp.float32).reshape(B, 1, L)
    biou2 = b_iou.reshape(1, 3 * H)
    bf2 = b_f.reshape(1, H)
    b1_2 = b1.reshape(1, HD)
    W2p = jnp.pad(W2, ((0, 0), (0, _OPAD - _OUT)))
    b2p = jnp.pad(b2, (0, _OPAD - _OUT)).reshape(1, _OPAD)
    UfUiou = jnp.concatenate([U_f, U_iou], axis=1).astype(jnp.bfloat16)

    const = lambda b: (0, 0)
    per_b3 = lambda b: (b, 0, 0)
    in_specs = [
        pl.BlockSpec((1, L, D), per_b3),
        pl.BlockSpec((1, L, D), per_b3),
        pl.BlockSpec((1, 1, L), per_b3),
        pl.BlockSpec((1, 1, 1), per_b3),
        pl.BlockSpec((1, 1, L), per_b3),
        pl.BlockSpec((D, 3 * H), const),
        pl.BlockSpec((1, 3 * H), const),
        pl.BlockSpec((D, H), const),
        pl.BlockSpec((H, 4 * H), const),
        pl.BlockSpec((1, H), const),
        pl.BlockSpec((3 * H + 2 * D, HD), const),
        pl.BlockSpec((1, HD), const),
        pl.BlockSpec((HD, _OPAD), const),
        pl.BlockSpec((1, _OPAD), const),
    ]
    out = pl.pallas_call(
        _body,
        grid=(B,),
        in_specs=in_specs,
        out_specs=pl.BlockSpec((1, _U * _U, _OPAD), per_b3),
        out_shape=jax.ShapeDtypeStruct((B, _U * _U, _OPAD), jnp.float32),
        compiler_params=pltpu.CompilerParams(
            dimension_semantics=("parallel",),
            allow_input_fusion=[False] * 2 + [True] * 3 + [False] * 3 + [True] * 1
            + [False] * 1 + [False] * 2 + [True] * 2),
    )(node_embs, token_embs, deps3, roots3, mask3,
      W_iou, biou2, W_f, UfUiou, bf2, W1, b1_2, W2p, b2p)
    return out[:, :, :_OUT].reshape(B, _U, _U, _OUT)


# speedup vs baseline: 1.5273x; 1.2628x over previous
import jax
import jax.numpy as jnp
from jax.experimental import pallas as pl
from jax.experimental.pallas import tpu as pltpu

_B, _L, _D, _H, _U, _OUT, _HD, _K = 16, 1024, 256, 256, 8, 3, 512, 6
_SPAN = _L // _U
_OPAD = 128


def _mm(a, b):
    return jax.lax.dot_general(a, b, (((1,), (0,)), ((), ())),
                               preferred_element_type=jnp.float32)


def _mmT(a, b):
    return jax.lax.dot_general(a, b, (((0,), (0,)), ((), ())),
                               preferred_element_type=jnp.float32)


def _sig(x):
    return 0.5 * jnp.tanh(0.5 * x) + 0.5


def _body(x_ref, te_ref, deps_ref, roots_ref, mask_ref,
          Wiou_ref, biou_ref, Wf_ref, UfUiou_ref, bf_ref,
          W1_ref, b1_ref, W2_ref, b2_ref, out_ref):
    f32 = jnp.float32
    bf16 = jnp.bfloat16
    L, H, D, HD, U, K = _L, _H, _D, _HD, _U, _K

    x = x_ref[0]
    te = te_ref[0]
    deps = deps_ref[0]
    mask = mask_ref[0]
    root = roots_ref[0]

    Wiou = Wiou_ref[...]
    biou = biou_ref[...]
    Wf = Wf_ref[...]
    bf = bf_ref[...]
    UfUiou = UfUiou_ref[...]
    Uf = UfUiou[:, :H]
    Uiou = UfUiou[:, H:]
    W1 = W1_ref[...]
    b1 = b1_ref[...]
    W2 = W2_ref[...]
    b2 = b2_ref[...]

    iota_row = jax.lax.broadcasted_iota(jnp.int32, (1, L), 1)
    v_row = ((deps != iota_row) & (mask > 0.5)).astype(f32)
    row_ids = jax.lax.broadcasted_iota(jnp.int32, (L, L), 0)
    Tv = ((row_ids == deps).astype(f32) * v_row).astype(bf16)
    valid_col = _mmT(Tv, jnp.ones((L, 1), bf16))

    xiou = _mm(x, Wiou) + biou
    xf = _mm(x, Wf) + bf
    xf_dst = _mmT(Tv, xf.astype(bf16))

    def _gates(iou, fc):
        i = _sig(iou[:, :H])
        o = _sig(iou[:, H:2 * H])
        u = jnp.tanh(iou[:, 2 * H:])
        c = i * u + fc
        h = o * jnp.tanh(c)
        return h, c

    def bu_round(hb, cb):
        hb_bf = hb.astype(bf16)
        f_b = _sig(xf_dst + _mm(hb_bf, Uf) * valid_col)
        sc = _mm(Tv, jnp.concatenate([hb_bf, (f_b * cb).astype(bf16)], axis=1))
        iou_b = xiou + _mm(sc[:, :H].astype(bf16), Uiou)
        return _gates(iou_b, sc[:, H:])

    h1, c1 = _gates(xiou, jnp.zeros((L, H), f32))

    hb, cb = h1, c1
    for _ in range(K - 2):
        hb, cb = bu_round(hb, cb)

    r_oh = (iota_row == root).astype(f32)
    hb_bf = hb.astype(bf16)
    f_b = _sig(xf_dst + _mm(hb_bf, Uf) * valid_col)
    A = jnp.concatenate([hb_bf, (f_b * cb).astype(bf16)], axis=1)
    w_row = ((deps == root).astype(f32) * v_row).astype(bf16)
    sc_r = _mm(w_row, A)
    iou_r = _mm(r_oh, xiou) + _mm(sc_r[:, :H].astype(bf16), Uiou)
    HpA, _ = _gates(iou_r, sc_r[:, H:])

    kk16 = jax.lax.broadcasted_iota(jnp.int32, (2 * U, L), 0)
    jj16 = jax.lax.broadcasted_iota(jnp.int32, (2 * U, L), 1)
    P16 = (jj16 == jnp.where(kk16 < U, kk16 * _SPAN,
                             (kk16 - U) * _SPAN + (_SPAN - 1))).astype(f32)
    jj16f = jj16.astype(f32)
    jcol = jax.lax.broadcasted_iota(jnp.int32, (L, 2), 1)
    icol = jax.lax.broadcasted_iota(jnp.int32, (L, 2), 0)
    jparts = jnp.where(jcol == 0, icol // 4, icol % 4).astype(bf16)
    depP = _mmT(Tv, jparts)
    Es = [P16]
    E = P16
    for _ in range(K - 1):
        d2 = _mm(E, depP)
        dep_at = 4.0 * d2[:, :1] + d2[:, 1:]
        E = (jj16f == dep_at).astype(f32)
        Es.append(E)

    h_rows = _mm(Es[K - 1], h1)
    c_rows = _mm(Es[K - 1], c1)
    for r in range(2, K + 1):
        Er = Es[K - r]
        m = _mm(Er, valid_col)
        xiou_r = _mm(Er, xiou)
        xf_r = _mm(Er, xf)
        hp = (h_rows * m).astype(bf16)
        fusedr = _mm(hp, UfUiou)
        f_t = _sig(xf_r + fusedr[:, :H])
        h_rows, c_rows = _gates(xiou_r + fusedr[:, H:], f_t * (c_rows * m))
    Hp1 = h_rows[:U]
    Hp2 = h_rows[U:]
    kk = jax.lax.broadcasted_iota(jnp.int32, (U, L), 0)
    jj = jax.lax.broadcasted_iota(jnp.int32, (U, L), 1)
    M8 = ((jj // _SPAN) == kk).astype(f32) * (1.0 / _SPAN)
    avg = _mm(M8, te)

    g0 = _mm(HpA, W1[0:H]) + b1
    G1 = _mm(Hp1, W1[H:2 * H])
    G2 = _mm(Hp2, W1[2 * H:3 * H])
    A1 = _mm(avg, W1[3 * H:3 * H + D])
    A2 = _mm(avg, W1[3 * H + D:3 * H + 2 * D])

    rowt = (G1 + A1).reshape(U, 1, HD)
    colt = (G2 + A2).reshape(1, U, HD)
    hid = jnp.tanh(rowt + colt + g0.reshape(1, 1, HD))
    out = _mm(hid.reshape(U * U, HD), W2) + b2
    out_ref[0] = out


def kernel(node_embs, token_embs, dependencies, roots, token_mask,
           W_iou, U_iou, b_iou, W_f, U_f, b_f, W1, b1, W2, b2):
    B, L, D, H, HD = _B, _L, _D, _H, _HD
    deps3 = dependencies.astype(jnp.int32).reshape(B, 1, L)
    roots3 = roots.astype(jnp.int32).reshape(B, 1, 1)
    mask3 = token_mask.astype(jnp.float32).reshape(B, 1, L)
    biou2 = b_iou.reshape(1, 3 * H)
    bf2 = b_f.reshape(1, H)
    b1_2 = b1.reshape(1, HD)
    W2p = jnp.pad(W2, ((0, 0), (0, _OPAD - _OUT)))
    b2p = jnp.pad(b2, (0, _OPAD - _OUT)).reshape(1, _OPAD)
    UfUiou = jnp.concatenate([U_f, U_iou], axis=1).astype(jnp.bfloat16)

    const = lambda b: (0, 0)
    per_b3 = lambda b: (b, 0, 0)
    in_specs = [
        pl.BlockSpec((1, L, D), per_b3),
        pl.BlockSpec((1, L, D), per_b3),
        pl.BlockSpec((1, 1, L), per_b3),
        pl.BlockSpec((1, 1, 1), per_b3),
        pl.BlockSpec((1, 1, L), per_b3),
        pl.BlockSpec((D, 3 * H), const),
        pl.BlockSpec((1, 3 * H), const),
        pl.BlockSpec((D, H), const),
        pl.BlockSpec((H, 4 * H), const),
        pl.BlockSpec((1, H), const),
        pl.BlockSpec((3 * H + 2 * D, HD), const),
        pl.BlockSpec((1, HD), const),
        pl.BlockSpec((HD, _OPAD), const),
        pl.BlockSpec((1, _OPAD), const),
    ]
    out = pl.pallas_call(
        _body,
        grid=(B,),
        in_specs=in_specs,
        out_specs=pl.BlockSpec((1, _U * _U, _OPAD), per_b3),
        out_shape=jax.ShapeDtypeStruct((B, _U * _U, _OPAD), jnp.float32),
        compiler_params=pltpu.CompilerParams(
            dimension_semantics=("parallel",),
            allow_input_fusion=[False] * 2 + [True] * 3 + [False] * 3 + [True] * 1
            + [False] * 1 + [False] * 2 + [True] * 2),
    )(node_embs, token_embs, deps3, roots3, mask3,
      W_iou, biou2, W_f, UfUiou, bf2, W1, b1_2, W2p, b2p)
    return out[:, :, :_OUT].reshape(B, _U, _U, _OUT)


# speedup vs baseline: 1.5317x; 1.0029x over previous
import jax
import jax.numpy as jnp
from jax.experimental import pallas as pl
from jax.experimental.pallas import tpu as pltpu

_B, _L, _D, _H, _U, _OUT, _HD, _K = 16, 1024, 256, 256, 8, 3, 512, 6
_SPAN = _L // _U
_OPAD = 128
_EPB = 2


def _mm(a, b):
    return jax.lax.dot_general(a, b, (((1,), (0,)), ((), ())),
                               preferred_element_type=jnp.float32)


def _mmT(a, b):
    return jax.lax.dot_general(a, b, (((0,), (0,)), ((), ())),
                               preferred_element_type=jnp.float32)


def _sig(x):
    return 0.5 * jnp.tanh(0.5 * x) + 0.5


def _body(x_ref, te_ref, deps_ref, roots_ref, mask_ref,
          Wiou_ref, biou_ref, Wf_ref, UfUiou_ref, bf_ref,
          W1_ref, b1_ref, W2_ref, b2_ref, out_ref):
    f32 = jnp.float32
    bf16 = jnp.bfloat16
    L, H, D, HD, U, K = _L, _H, _D, _HD, _U, _K

    Wiou = Wiou_ref[...]
    biou = biou_ref[...]
    Wf = Wf_ref[...]
    bf = bf_ref[...]
    UfUiou = UfUiou_ref[...]
    Uf = UfUiou[:, :H]
    Uiou = UfUiou[:, H:]
    W1 = W1_ref[...]
    b1 = b1_ref[...]
    W2 = W2_ref[...]
    b2 = b2_ref[...]

    def _gates(iou, fc):
        i = _sig(iou[:, :H])
        o = _sig(iou[:, H:2 * H])
        u = jnp.tanh(iou[:, 2 * H:])
        c = i * u + fc
        h = o * jnp.tanh(c)
        return h, c

    def _one(e):
        x = x_ref[e]
        te = te_ref[e]
        deps = deps_ref[e]
        mask = mask_ref[e]
        root = roots_ref[e]

        iota_row = jax.lax.broadcasted_iota(jnp.int32, (1, L), 1)
        v_row = ((deps != iota_row) & (mask > 0.5)).astype(f32)
        row_ids = jax.lax.broadcasted_iota(jnp.int32, (L, L), 0)
        Tv = ((row_ids == deps).astype(f32) * v_row).astype(bf16)
        valid_col = _mmT(Tv, jnp.ones((L, 1), bf16))

        xiou = _mm(x, Wiou) + biou
        xf = _mm(x, Wf) + bf
        xf_dst = _mmT(Tv, xf.astype(bf16))

        def bu_round(hb, cb):
            hb_bf = hb.astype(bf16)
            f_b = _sig(xf_dst + _mm(hb_bf, Uf) * valid_col)
            sc = _mm(Tv, jnp.concatenate([hb_bf, (f_b * cb).astype(bf16)],
                                         axis=1))
            iou_b = xiou + _mm(sc[:, :H].astype(bf16), Uiou)
            return _gates(iou_b, sc[:, H:])

        h1, c1 = _gates(xiou, jnp.zeros((L, H), f32))

        hb, cb = h1, c1
        for _ in range(K - 2):
            hb, cb = bu_round(hb, cb)

        r_oh = (iota_row == root).astype(f32)
        hb_bf = hb.astype(bf16)
        f_b = _sig(xf_dst + _mm(hb_bf, Uf) * valid_col)
        A = jnp.concatenate([hb_bf, (f_b * cb).astype(bf16)], axis=1)
        w_row = ((deps == root).astype(f32) * v_row).astype(bf16)
        sc_r = _mm(w_row, A)
        iou_r = _mm(r_oh, xiou) + _mm(sc_r[:, :H].astype(bf16), Uiou)
        HpA, _ = _gates(iou_r, sc_r[:, H:])

        kk16 = jax.lax.broadcasted_iota(jnp.int32, (2 * U, L), 0)
        jj16 = jax.lax.broadcasted_iota(jnp.int32, (2 * U, L), 1)
        P16 = (jj16 == jnp.where(kk16 < U, kk16 * _SPAN,
                                 (kk16 - U) * _SPAN + (_SPAN - 1))).astype(f32)
        jj16f = jj16.astype(f32)
        jcol = jax.lax.broadcasted_iota(jnp.int32, (L, 2), 1)
        icol = jax.lax.broadcasted_iota(jnp.int32, (L, 2), 0)
        jparts = jnp.where(jcol == 0, icol // 4, icol % 4).astype(bf16)
        depP = _mmT(Tv, jparts)
        Es = [P16]
        E = P16
        for _ in range(K - 1):
            d2 = _mm(E, depP)
            dep_at = 4.0 * d2[:, :1] + d2[:, 1:]
            E = (jj16f == dep_at).astype(f32)
            Es.append(E)

        h_rows = _mm(Es[K - 1], h1)
        c_rows = _mm(Es[K - 1], c1)
        for r in range(2, K + 1):
            Er = Es[K - r]
            m = _mm(Er, valid_col)
            xiou_r = _mm(Er, xiou)
            xf_r = _mm(Er, xf)
            hp = (h_rows * m).astype(bf16)
            fusedr = _mm(hp, UfUiou)
            f_t = _sig(xf_r + fusedr[:, :H])
            h_rows, c_rows = _gates(xiou_r + fusedr[:, H:],
                                    f_t * (c_rows * m))
        Hp1 = h_rows[:U]
        Hp2 = h_rows[U:]
        kk = jax.lax.broadcasted_iota(jnp.int32, (U, L), 0)
        jj = jax.lax.broadcasted_iota(jnp.int32, (U, L), 1)
        M8 = ((jj // _SPAN) == kk).astype(f32) * (1.0 / _SPAN)
        avg = _mm(M8, te)

        g0 = _mm(HpA, W1[0:H]) + b1
        G1 = _mm(Hp1, W1[H:2 * H])
        G2 = _mm(Hp2, W1[2 * H:3 * H])
        A1 = _mm(avg, W1[3 * H:3 * H + D])
        A2 = _mm(avg, W1[3 * H + D:3 * H + 2 * D])

        rowt = (G1 + A1).reshape(U, 1, HD)
        colt = (G2 + A2).reshape(1, U, HD)
        hid = jnp.tanh(rowt + colt + g0.reshape(1, 1, HD))
        out = _mm(hid.reshape(U * U, HD), W2) + b2
        out_ref[e] = out

    for e in range(_EPB):
        _one(e)


def kernel(node_embs, token_embs, dependencies, roots, token_mask,
           W_iou, U_iou, b_iou, W_f, U_f, b_f, W1, b1, W2, b2):
    B, L, D, H, HD, E = _B, _L, _D, _H, _HD, _EPB
    deps3 = dependencies.astype(jnp.int32).reshape(B, 1, L)
    roots3 = roots.astype(jnp.int32).reshape(B, 1, 1)
    mask3 = token_mask.astype(jnp.float32).reshape(B, 1, L)
    biou2 = b_iou.reshape(1, 3 * H)
    bf2 = b_f.reshape(1, H)
    b1_2 = b1.reshape(1, HD)
    W2p = jnp.pad(W2, ((0, 0), (0, _OPAD - _OUT)))
    b2p = jnp.pad(b2, (0, _OPAD - _OUT)).reshape(1, _OPAD)
    UfUiou = jnp.concatenate([U_f, U_iou], axis=1).astype(jnp.bfloat16)

    const = lambda b: (0, 0)
    per_b3 = lambda b: (b, 0, 0)
    in_specs = [
        pl.BlockSpec((E, L, D), per_b3),
        pl.BlockSpec((E, L, D), per_b3),
        pl.BlockSpec((E, 1, L), per_b3),
        pl.BlockSpec((E, 1, 1), per_b3),
        pl.BlockSpec((E, 1, L), per_b3),
        pl.BlockSpec((D, 3 * H), const),
        pl.BlockSpec((1, 3 * H), const),
        pl.BlockSpec((D, H), const),
        pl.BlockSpec((H, 4 * H), const),
        pl.BlockSpec((1, H), const),
        pl.BlockSpec((3 * H + 2 * D, HD), const),
        pl.BlockSpec((1, HD), const),
        pl.BlockSpec((HD, _OPAD), const),
        pl.BlockSpec((1, _OPAD), const),
    ]
    out = pl.pallas_call(
        _body,
        grid=(B // E,),
        in_specs=in_specs,
        out_specs=pl.BlockSpec((E, _U * _U, _OPAD), per_b3),
        out_shape=jax.ShapeDtypeStruct((B, _U * _U, _OPAD), jnp.float32),
        compiler_params=pltpu.CompilerParams(
            dimension_semantics=("parallel",),
            allow_input_fusion=[False] * 2 + [True] * 3 + [False] * 3 + [True] * 1
            + [False] * 1 + [False] * 2 + [True] * 2),
    )(node_embs, token_embs, deps3, roots3, mask3,
      W_iou, biou2, W_f, UfUiou, bf2, W1, b1_2, W2p, b2p)
    return out[:, :, :_OUT].reshape(B, _U, _U, _OUT)
